# baseline (speedup 1.0000x reference)
import functools

import jax
import jax.numpy as jnp
from jax import lax
from jax.experimental import pallas as pl
from jax.experimental.pallas import tpu as pltpu
from jax.experimental.pallas import tpu_sc as plsc

NC = 2
NS = 16
CH = 80
BG = 128
RB = 400


def _sc_mesh():
    return plsc.VectorSubcoreMesh(
        core_axis_name="c", subcore_axis_name="s", num_cores=NC, num_subcores=NS
    )


def _rowwise_copy(s, n, src_at, dst_at):
    rm = (n // (8 * NS)) * 8
    rem = n - NS * rm
    base = s * rm
    pltpu.sync_copy(src_at(base, rm), dst_at(base, rm))
    if rem:
        @pl.when(s == NS - 1)
        def _():
            pltpu.sync_copy(src_at(NS * rm, rem), dst_at(NS * rm, rem))


def _deg_body(n, e, dst_hbm, zeros_hbm, ones_hbm, out_hbm,
              dstA, dstB, ones_v, table, isemA, isemB, ssemA, ssemB):
    c = lax.axis_index("c")
    s = lax.axis_index("s")
    _rowwise_copy(s, n, lambda b, r: zeros_hbm.at[pl.ds(b, r)],
                  lambda b, r: table.at[pl.ds(b, r)])
    pltpu.sync_copy(ones_hbm, ones_v)
    plsc.subcore_barrier()
    per_tile = e // (NC * NS)
    base = c * (e // NC) + s * per_tile
    nch = per_tile // CH

    dsts = (dstA, dstB)
    isems = (isemA, isemB)
    ssems = (ssemA, ssemB)

    def idx_start(j, p):
        off = base + j * CH
        pltpu.async_copy(dst_hbm.at[pl.ds(off, CH)], dsts[p], isems[p])

    def idx_wait(j, p):
        off = base + j * CH
        pltpu.make_async_copy(dst_hbm.at[pl.ds(off, CH)], dsts[p], isems[p]).wait()

    def step(t, b, k):
        q = 1 - b
        @pl.when((t >= 1) & (t - 1 < nch))
        def _():
            idx_wait(t - 1, q)
            pltpu.async_copy(ones_v, table.at[dsts[q]], ssems[q], add=True)
        @pl.when(k >= 1)
        def _():
            pltpu.make_async_copy(ones_v, table.at[dsts[b]], ssems[b]).wait()
        @pl.when(t < nch)
        def _():
            idx_start(t, b)

    def body(k, carry):
        step(2 * k, 0, k)
        step(2 * k + 1, 1, k)
        return carry

    nsteps = nch + 1
    lax.fori_loop(0, nsteps // 2, body, 0)
    for t in range((nsteps // 2) * 2, nsteps):
        step(t, t % 2, 1)
    pltpu.make_async_copy(ones_v, table.at[dsts[(nch - 1) % 2]],
                          ssems[(nch - 1) % 2]).wait()
    plsc.subcore_barrier()
    _rowwise_copy(s, n, lambda b, r: table.at[pl.ds(b, r)],
                  lambda b, r: out_hbm.at[c, pl.ds(b, r)])


def _deg_call(dst, zeros, n, e):
    body = functools.partial(_deg_body, n, e)
    k = pl.kernel(
        body,
        out_type=jax.ShapeDtypeStruct((NC, n, 128), jnp.float32),
        mesh=_sc_mesh(),
        scratch_types=[
            pltpu.VMEM((CH,), jnp.int32),
            pltpu.VMEM((CH,), jnp.int32),
            pltpu.VMEM((CH, 128), jnp.float32),
            pltpu.VMEM_SHARED((n, 128), jnp.float32),
            pltpu.SemaphoreType.DMA,
            pltpu.SemaphoreType.DMA,
            pltpu.SemaphoreType.DMA,
            pltpu.SemaphoreType.DMA,
        ],
    )
    return k(dst, zeros, jnp.ones((CH, 128), jnp.float32))


def _edge_body(n, e, hs_hbm, src_hbm, dst_hbm, zeros_hbm, out_hbm,
               srcs, dsts, rows, accum, isems, gsems, ssems):
    c = lax.axis_index("c")
    s = lax.axis_index("s")
    _rowwise_copy(s, n, lambda b, r: zeros_hbm.at[pl.ds(b, r)],
                  lambda b, r: accum.at[pl.ds(b, r)])
    plsc.subcore_barrier()
    per_tile = e // (NC * NS)
    base = c * (e // NC) + s * per_tile
    nch = per_tile // CH

    def idx_start(j, p):
        off = base + j * CH
        pltpu.async_copy(src_hbm.at[pl.ds(off, CH)], srcs[p], isems[p])
        pltpu.async_copy(dst_hbm.at[pl.ds(off, CH)], dsts[p], isems[p])

    def idx_wait(j, p):
        off = base + j * CH
        pltpu.make_async_copy(src_hbm.at[pl.ds(off, CH)], srcs[p], isems[p]).wait()
        pltpu.make_async_copy(dst_hbm.at[pl.ds(off, CH)], dsts[p], isems[p]).wait()

    def gather_start(p):
        pltpu.async_copy(hs_hbm.at[srcs[p]], rows[p], gsems[p])

    def gather_wait(p):
        pltpu.make_async_copy(hs_hbm.at[srcs[p]], rows[p], gsems[p]).wait()

    def scatter_start(p):
        pltpu.async_copy(rows[p], accum.at[dsts[p]], ssems[p], add=True)

    def scatter_wait(p):
        pltpu.make_async_copy(rows[p], accum.at[dsts[p]], ssems[p]).wait()

    def step(t, b, k):
        q = (b - 1) % 3
        @pl.when((t >= 1) & (t - 1 < nch))
        def _():
            idx_wait(t - 1, q)
            gather_start(q)
        r_ = (b - 2) % 3
        @pl.when((t >= 2) & (t - 2 < nch))
        def _():
            gather_wait(r_)
            scatter_start(r_)
        @pl.when(k >= 1)
        def _():
            scatter_wait(b)
        @pl.when(t < nch)
        def _():
            idx_start(t, b)

    def body(k, carry):
        for b in range(3):
            step(3 * k + b, b, k)
        return carry

    nsteps = nch + 2
    lax.fori_loop(0, nsteps // 3, body, 0)
    for t in range((nsteps // 3) * 3, nsteps):
        step(t, t % 3, 1)
    scatter_wait((nch - 1) % 3)
    plsc.subcore_barrier()
    _rowwise_copy(s, n, lambda b, r: accum.at[pl.ds(b, r)],
                  lambda b, r: out_hbm.at[c, pl.ds(b, r)])


def _edge_call(hs, src, dst, zeros, n, e):
    assert (e // (NC * NS)) % CH == 0 and (e // (NC * NS)) // CH >= 6
    body = functools.partial(_edge_body, n, e)
    k = pl.kernel(
        body,
        out_type=jax.ShapeDtypeStruct((NC, n, 128), jnp.float32),
        mesh=_sc_mesh(),
        scratch_types=[
            [pltpu.VMEM((CH,), jnp.int32)] * 3,
            [pltpu.VMEM((CH,), jnp.int32)] * 3,
            [pltpu.VMEM((CH, 128), jnp.float32)] * 3,
            pltpu.VMEM_SHARED((n, 128), jnp.float32),
            [pltpu.SemaphoreType.DMA] * 3,
            [pltpu.SemaphoreType.DMA] * 3,
            [pltpu.SemaphoreType.DMA] * 3,
        ],
    )
    return k(hs, src, dst, zeros)


def _dinv(t0_ref, t1_ref):
    deg = t0_ref[:, 0:1] + t1_ref[:, 0:1] + 1.0
    return lax.rsqrt(jnp.maximum(deg, 1.0))


def _mm_scale_body(x_ref, w_ref, t0_ref, t1_ref, o_ref):
    h = jnp.dot(x_ref[...], w_ref[...], preferred_element_type=jnp.float32)
    o_ref[...] = h * _dinv(t0_ref, t1_ref)


def _mm_scale(x, w, t0, t1, n):
    grid = (n // RB,)
    return pl.pallas_call(
        _mm_scale_body,
        grid=grid,
        in_specs=[
            pl.BlockSpec((RB, 128), lambda i: (i, 0)),
            pl.BlockSpec((128, 128), lambda i: (0, 0)),
            pl.BlockSpec((RB, 128), lambda i: (i, 0)),
            pl.BlockSpec((RB, 128), lambda i: (i, 0)),
        ],
        out_specs=pl.BlockSpec((RB, 128), lambda i: (i, 0)),
        out_shape=jax.ShapeDtypeStruct((n, 128), jnp.float32),
    )(x, w, t0, t1)


def _layer_body(a0_ref, a1_ref, hs_ref, t0_ref, t1_ref, b_ref, w_ref, o_ref):
    dinv = _dinv(t0_ref, t1_ref)
    h = dinv * (a0_ref[...] + a1_ref[...] + hs_ref[...]) + b_ref[...]
    h = jnp.maximum(h, 0.0)
    o_ref[...] = jnp.dot(h, w_ref[...], preferred_element_type=jnp.float32) * dinv


def _layer(a0, a1, hs, t0, t1, b, w, n):
    grid = (n // RB,)
    return pl.pallas_call(
        _layer_body,
        grid=grid,
        in_specs=[
            pl.BlockSpec((RB, 128), lambda i: (i, 0)),
            pl.BlockSpec((RB, 128), lambda i: (i, 0)),
            pl.BlockSpec((RB, 128), lambda i: (i, 0)),
            pl.BlockSpec((RB, 128), lambda i: (i, 0)),
            pl.BlockSpec((RB, 128), lambda i: (i, 0)),
            pl.BlockSpec((1, 128), lambda i: (0, 0)),
            pl.BlockSpec((128, 128), lambda i: (0, 0)),
        ],
        out_specs=pl.BlockSpec((RB, 128), lambda i: (i, 0)),
        out_shape=jax.ShapeDtypeStruct((n, 128), jnp.float32),
    )(a0, a1, hs, t0, t1, b, w)


def _pool_body(nblk, a0_ref, a1_ref, hs_ref, t0_ref, t1_ref, b_ref, batch_ref,
               wc_ref, bc_ref, o_ref, sums, cnts):
    i = pl.program_id(0)
    dinv = _dinv(t0_ref, t1_ref)
    h = dinv * (a0_ref[...] + a1_ref[...] + hs_ref[...]) + b_ref[...]
    h = jnp.maximum(h, 0.0)
    gids = lax.broadcasted_iota(jnp.int32, (RB, BG), 1)
    onehot = (batch_ref[...] == gids).astype(jnp.float32)
    dn = (((0,), (0,)), ((), ()))
    ps = lax.dot_general(onehot, h, dn, preferred_element_type=jnp.float32,
                         precision=lax.Precision.HIGHEST)
    pc = lax.dot_general(onehot, jnp.ones_like(h), dn,
                         preferred_element_type=jnp.float32)

    @pl.when(i == 0)
    def _():
        sums[...] = ps
        cnts[...] = pc

    @pl.when(i > 0)
    def _():
        sums[...] += ps
        cnts[...] += pc

    @pl.when(i == nblk - 1)
    def _():
        pooled = sums[...] / jnp.maximum(cnts[...], 1.0)
        o_ref[...] = jnp.dot(pooled, wc_ref[...],
                             preferred_element_type=jnp.float32) + bc_ref[...]


def _pool(a0, a1, hs, t0, t1, b, batch2d, wc, bc, n):
    nblk = n // RB
    return pl.pallas_call(
        functools.partial(_pool_body, nblk),
        grid=(nblk,),
        in_specs=[
            pl.BlockSpec((RB, 128), lambda i: (i, 0)),
            pl.BlockSpec((RB, 128), lambda i: (i, 0)),
            pl.BlockSpec((RB, 128), lambda i: (i, 0)),
            pl.BlockSpec((RB, 128), lambda i: (i, 0)),
            pl.BlockSpec((RB, 128), lambda i: (i, 0)),
            pl.BlockSpec((1, 128), lambda i: (0, 0)),
            pl.BlockSpec((RB, 1), lambda i: (i, 0)),
            pl.BlockSpec((128, 1), lambda i: (0, 0)),
            pl.BlockSpec((1, 1), lambda i: (0, 0)),
        ],
        out_specs=pl.BlockSpec((BG, 1), lambda i: (0, 0)),
        out_shape=jax.ShapeDtypeStruct((BG, 1), jnp.float32),
        scratch_shapes=[
            pltpu.VMEM((BG, 128), jnp.float32),
            pltpu.VMEM((BG, 128), jnp.float32),
        ],
    )(a0, a1, hs, t0, t1, b, batch2d, wc, bc)


def kernel(x, edge_index, batch, W1, b1, W2, b2, Wc, bc):
    n, d = x.shape
    e = edge_index.shape[1]
    src = edge_index[0]
    dst = edge_index[1]
    zeros = jnp.zeros((n, 128), jnp.float32)

    degt = _deg_call(dst, zeros, n, e)
    t0, t1 = degt[0], degt[1]

    hs1 = _mm_scale(x, W1, t0, t1, n)
    acc1 = _edge_call(hs1, src, dst, zeros, n, e)
    hs2 = _layer(acc1[0], acc1[1], hs1, t0, t1, b1.reshape(1, -1), W2, n)
    acc2 = _edge_call(hs2, src, dst, zeros, n, e)
    out = _pool(acc2[0], acc2[1], hs2, t0, t1, b2.reshape(1, -1),
                batch.reshape(-1, 1), Wc, bc.reshape(1, 1), n)
    return out


# speedup vs baseline: 1.1220x; 1.1220x over previous
import functools

import jax
import jax.numpy as jnp
from jax import lax
from jax.experimental import pallas as pl
from jax.experimental.pallas import tpu as pltpu
from jax.experimental.pallas import tpu_sc as plsc

NC = 2
NS = 16
CH = 80
BG = 128
RB = 400


def _sc_mesh():
    return plsc.VectorSubcoreMesh(
        core_axis_name="c", subcore_axis_name="s", num_cores=NC, num_subcores=NS
    )


def _rowwise_copy(s, n, src_at, dst_at):
    rm = (n // (8 * NS)) * 8
    rem = n - NS * rm
    base = s * rm
    pltpu.sync_copy(src_at(base, rm), dst_at(base, rm))
    if rem:
        @pl.when(s == NS - 1)
        def _():
            pltpu.sync_copy(src_at(NS * rm, rem), dst_at(NS * rm, rem))


def _deg_body(n, e, dst_hbm, zeros_hbm, ones_hbm, out_hbm,
              dstA, dstB, ones_v, table, isemA, isemB, ssemA, ssemB):
    c = lax.axis_index("c")
    s = lax.axis_index("s")
    _rowwise_copy(s, n, lambda b, r: zeros_hbm.at[pl.ds(b, r)],
                  lambda b, r: table.at[pl.ds(b, r)])
    pltpu.sync_copy(ones_hbm, ones_v)
    plsc.subcore_barrier()
    per_tile = e // (NC * NS)
    base = c * (e // NC) + s * per_tile
    nch = per_tile // CH

    dsts = (dstA, dstB)
    isems = (isemA, isemB)
    ssems = (ssemA, ssemB)

    def idx_start(j, p):
        off = base + j * CH
        pltpu.async_copy(dst_hbm.at[pl.ds(off, CH)], dsts[p], isems[p])

    def idx_wait(j, p):
        off = base + j * CH
        pltpu.make_async_copy(dst_hbm.at[pl.ds(off, CH)], dsts[p], isems[p]).wait()

    def step(t, b, k):
        q = 1 - b
        @pl.when((t >= 1) & (t - 1 < nch))
        def _():
            idx_wait(t - 1, q)
            pltpu.async_copy(ones_v, table.at[dsts[q]], ssems[q], add=True)
        @pl.when(k >= 1)
        def _():
            pltpu.make_async_copy(ones_v, table.at[dsts[b]], ssems[b]).wait()
        @pl.when(t < nch)
        def _():
            idx_start(t, b)

    def body(k, carry):
        step(2 * k, 0, k)
        step(2 * k + 1, 1, k)
        return carry

    nsteps = nch + 1
    lax.fori_loop(0, nsteps // 2, body, 0)
    for t in range((nsteps // 2) * 2, nsteps):
        step(t, t % 2, 1)
    pltpu.make_async_copy(ones_v, table.at[dsts[(nch - 1) % 2]],
                          ssems[(nch - 1) % 2]).wait()
    plsc.subcore_barrier()
    _rowwise_copy(s, n, lambda b, r: table.at[pl.ds(b, r)],
                  lambda b, r: out_hbm.at[c, pl.ds(b, r)])


def _deg_call(dst, zeros, n, e):
    body = functools.partial(_deg_body, n, e)
    k = pl.kernel(
        body,
        out_type=jax.ShapeDtypeStruct((NC, n, 128), jnp.float32),
        mesh=_sc_mesh(),
        scratch_types=[
            pltpu.VMEM((CH,), jnp.int32),
            pltpu.VMEM((CH,), jnp.int32),
            pltpu.VMEM((CH, 128), jnp.float32),
            pltpu.VMEM_SHARED((n, 128), jnp.float32),
            pltpu.SemaphoreType.DMA,
            pltpu.SemaphoreType.DMA,
            pltpu.SemaphoreType.DMA,
            pltpu.SemaphoreType.DMA,
        ],
    )
    return k(dst, zeros, jnp.ones((CH, 128), jnp.float32))


def _edge_body(n, e, hs_hbm, src_hbm, dst_hbm, zeros_hbm, out_hbm,
               srcs, dsts, rows, accum, isems, gsems, ssems):
    c = lax.axis_index("c")
    s = lax.axis_index("s")
    _rowwise_copy(s, n, lambda b, r: zeros_hbm.at[pl.ds(b, r)],
                  lambda b, r: accum.at[pl.ds(b, r)])
    plsc.subcore_barrier()
    per_tile = e // (NC * NS)
    base = c * (e // NC) + s * per_tile
    nch = per_tile // CH

    NSLOT = 4

    def idx_start(j, p):
        off = base + j * CH
        pltpu.async_copy(src_hbm.at[pl.ds(off, CH)], srcs[p], isems[p])
        pltpu.async_copy(dst_hbm.at[pl.ds(off, CH)], dsts[p], isems[p])

    def idx_wait(j, p):
        off = base + j * CH
        pltpu.make_async_copy(src_hbm.at[pl.ds(off, CH)], srcs[p], isems[p]).wait()
        pltpu.make_async_copy(dst_hbm.at[pl.ds(off, CH)], dsts[p], isems[p]).wait()

    def gather_start(p):
        pltpu.async_copy(hs_hbm.at[srcs[p]], rows[p], gsems[p])

    def gather_wait(p):
        pltpu.make_async_copy(hs_hbm.at[srcs[p]], rows[p], gsems[p]).wait()

    def scatter_start(p):
        pltpu.async_copy(rows[p], accum.at[dsts[p]], ssems[p], add=True)

    def scatter_wait(p):
        pltpu.make_async_copy(rows[p], accum.at[dsts[p]], ssems[p]).wait()

    def step(t, b, k):
        q = (b - 1) % NSLOT
        @pl.when((t >= 1) & (t - 1 < nch))
        def _():
            idx_wait(t - 1, q)
            gather_start(q)
        r_ = (b - 3) % NSLOT
        @pl.when((t >= 3) & (t - 3 < nch))
        def _():
            gather_wait(r_)
            scatter_start(r_)
        @pl.when(k >= 1)
        def _():
            scatter_wait(b)
        @pl.when(t < nch)
        def _():
            idx_start(t, b)

    def body(k, carry):
        for b in range(NSLOT):
            step(NSLOT * k + b, b, k)
        return carry

    nsteps = nch + 3
    lax.fori_loop(0, nsteps // NSLOT, body, 0)
    for t in range((nsteps // NSLOT) * NSLOT, nsteps):
        step(t, t % NSLOT, 1)
    scatter_wait((nch - 1) % NSLOT)
    plsc.subcore_barrier()
    _rowwise_copy(s, n, lambda b, r: accum.at[pl.ds(b, r)],
                  lambda b, r: out_hbm.at[c, pl.ds(b, r)])


def _edge_call(hs, src, dst, zeros, n, e):
    assert (e // (NC * NS)) % CH == 0 and (e // (NC * NS)) // CH >= 6
    body = functools.partial(_edge_body, n, e)
    k = pl.kernel(
        body,
        out_type=jax.ShapeDtypeStruct((NC, n, 128), jnp.float32),
        mesh=_sc_mesh(),
        scratch_types=[
            [pltpu.VMEM((CH,), jnp.int32)] * 4,
            [pltpu.VMEM((CH,), jnp.int32)] * 4,
            [pltpu.VMEM((CH, 128), jnp.float32)] * 4,
            pltpu.VMEM_SHARED((n, 128), jnp.float32),
            [pltpu.SemaphoreType.DMA] * 4,
            [pltpu.SemaphoreType.DMA] * 4,
            [pltpu.SemaphoreType.DMA] * 4,
        ],
    )
    return k(hs, src, dst, zeros)


def _dinv(t0_ref, t1_ref):
    deg = t0_ref[:, 0:1] + t1_ref[:, 0:1] + 1.0
    return lax.rsqrt(jnp.maximum(deg, 1.0))


def _mm_scale_body(x_ref, w_ref, t0_ref, t1_ref, o_ref):
    h = jnp.dot(x_ref[...], w_ref[...], preferred_element_type=jnp.float32)
    o_ref[...] = h * _dinv(t0_ref, t1_ref)


def _mm_scale(x, w, t0, t1, n):
    grid = (n // RB,)
    return pl.pallas_call(
        _mm_scale_body,
        grid=grid,
        in_specs=[
            pl.BlockSpec((RB, 128), lambda i: (i, 0)),
            pl.BlockSpec((128, 128), lambda i: (0, 0)),
            pl.BlockSpec((RB, 128), lambda i: (i, 0)),
            pl.BlockSpec((RB, 128), lambda i: (i, 0)),
        ],
        out_specs=pl.BlockSpec((RB, 128), lambda i: (i, 0)),
        out_shape=jax.ShapeDtypeStruct((n, 128), jnp.float32),
    )(x, w, t0, t1)


def _layer_body(a0_ref, a1_ref, hs_ref, t0_ref, t1_ref, b_ref, w_ref, o_ref):
    dinv = _dinv(t0_ref, t1_ref)
    h = dinv * (a0_ref[...] + a1_ref[...] + hs_ref[...]) + b_ref[...]
    h = jnp.maximum(h, 0.0)
    o_ref[...] = jnp.dot(h, w_ref[...], preferred_element_type=jnp.float32) * dinv


def _layer(a0, a1, hs, t0, t1, b, w, n):
    grid = (n // RB,)
    return pl.pallas_call(
        _layer_body,
        grid=grid,
        in_specs=[
            pl.BlockSpec((RB, 128), lambda i: (i, 0)),
            pl.BlockSpec((RB, 128), lambda i: (i, 0)),
            pl.BlockSpec((RB, 128), lambda i: (i, 0)),
            pl.BlockSpec((RB, 128), lambda i: (i, 0)),
            pl.BlockSpec((RB, 128), lambda i: (i, 0)),
            pl.BlockSpec((1, 128), lambda i: (0, 0)),
            pl.BlockSpec((128, 128), lambda i: (0, 0)),
        ],
        out_specs=pl.BlockSpec((RB, 128), lambda i: (i, 0)),
        out_shape=jax.ShapeDtypeStruct((n, 128), jnp.float32),
    )(a0, a1, hs, t0, t1, b, w)


def _pool_body(nblk, a0_ref, a1_ref, hs_ref, t0_ref, t1_ref, b_ref, batch_ref,
               wc_ref, bc_ref, o_ref, sums, cnts):
    i = pl.program_id(0)
    dinv = _dinv(t0_ref, t1_ref)
    h = dinv * (a0_ref[...] + a1_ref[...] + hs_ref[...]) + b_ref[...]
    h = jnp.maximum(h, 0.0)
    gids = lax.broadcasted_iota(jnp.int32, (RB, BG), 1)
    onehot = (batch_ref[...] == gids).astype(jnp.float32)
    dn = (((0,), (0,)), ((), ()))
    ps = lax.dot_general(onehot, h, dn, preferred_element_type=jnp.float32,
                         precision=lax.Precision.HIGHEST)
    pc = lax.dot_general(onehot, jnp.ones_like(h), dn,
                         preferred_element_type=jnp.float32)

    @pl.when(i == 0)
    def _():
        sums[...] = ps
        cnts[...] = pc

    @pl.when(i > 0)
    def _():
        sums[...] += ps
        cnts[...] += pc

    @pl.when(i == nblk - 1)
    def _():
        pooled = sums[...] / jnp.maximum(cnts[...], 1.0)
        o_ref[...] = jnp.dot(pooled, wc_ref[...],
                             preferred_element_type=jnp.float32) + bc_ref[...]


def _pool(a0, a1, hs, t0, t1, b, batch2d, wc, bc, n):
    nblk = n // RB
    return pl.pallas_call(
        functools.partial(_pool_body, nblk),
        grid=(nblk,),
        in_specs=[
            pl.BlockSpec((RB, 128), lambda i: (i, 0)),
            pl.BlockSpec((RB, 128), lambda i: (i, 0)),
            pl.BlockSpec((RB, 128), lambda i: (i, 0)),
            pl.BlockSpec((RB, 128), lambda i: (i, 0)),
            pl.BlockSpec((RB, 128), lambda i: (i, 0)),
            pl.BlockSpec((1, 128), lambda i: (0, 0)),
            pl.BlockSpec((RB, 1), lambda i: (i, 0)),
            pl.BlockSpec((128, 1), lambda i: (0, 0)),
            pl.BlockSpec((1, 1), lambda i: (0, 0)),
        ],
        out_specs=pl.BlockSpec((BG, 1), lambda i: (0, 0)),
        out_shape=jax.ShapeDtypeStruct((BG, 1), jnp.float32),
        scratch_shapes=[
            pltpu.VMEM((BG, 128), jnp.float32),
            pltpu.VMEM((BG, 128), jnp.float32),
        ],
    )(a0, a1, hs, t0, t1, b, batch2d, wc, bc)


def kernel(x, edge_index, batch, W1, b1, W2, b2, Wc, bc):
    n, d = x.shape
    e = edge_index.shape[1]
    src = edge_index[0]
    dst = edge_index[1]
    zeros = jnp.zeros((n, 128), jnp.float32)

    degt = _deg_call(dst, zeros, n, e)
    t0, t1 = degt[0], degt[1]

    hs1 = _mm_scale(x, W1, t0, t1, n)
    acc1 = _edge_call(hs1, src, dst, zeros, n, e)
    hs2 = _layer(acc1[0], acc1[1], hs1, t0, t1, b1.reshape(1, -1), W2, n)
    acc2 = _edge_call(hs2, src, dst, zeros, n, e)
    out = _pool(acc2[0], acc2[1], hs2, t0, t1, b2.reshape(1, -1),
                batch.reshape(-1, 1), Wc, bc.reshape(1, 1), n)
    return out
